# baseline (speedup 1.0000x reference)
import functools

import jax
import jax.numpy as jnp
from jax import lax
from jax.experimental import pallas as pl
from jax.experimental.pallas import tpu as pltpu
from jax.experimental.pallas import tpu_sc as plsc

N = 10000
NP = 10240
E = 320000
NWORK = 32
EW = E // NWORK
CH = 80
NCHUNK = EW // CH
NBUF = 5
NGRP = NCHUNK // NBUF


def _seg_sum_sc(table, src3, dst4, rows_out, feat, nbuf, bias=None,
                combine=False):
    mesh = plsc.VectorSubcoreMesh(core_axis_name="c", subcore_axis_name="s",
                                  num_cores=2, num_subcores=16)
    rs = rows_out // 16
    sem_names = ["g%d" % b for b in range(nbuf)]

    out_type = jax.ShapeDtypeStruct((2, rows_out, feat), jnp.float32)
    if combine:
        out_type = (out_type,
                    jax.ShapeDtypeStruct((2 * NP, feat), jnp.float32))

    @functools.partial(
        pl.kernel,
        mesh=mesh,
        out_type=out_type,
        scratch_types=dict(
            srcv=pltpu.VMEM((NCHUNK, CH), jnp.int32),
            dstv=pltpu.VMEM((NCHUNK, CH), jnp.int32),
            rowb=pltpu.VMEM((nbuf, CH, feat), jnp.float32),
            bv=pltpu.VMEM((64,), jnp.float32),
            acc=pltpu.VMEM_SHARED((rows_out, feat), jnp.float32),
            ssem=pltpu.SemaphoreType.DMA,
            **{nm: pltpu.SemaphoreType.DMA for nm in sem_names},
        ),
        compiler_params=pltpu.CompilerParams(use_tc_tiling_on_sc=False),
    )
    def k(*ins, srcv, dstv, rowb, bv, acc, ssem, **kw):
        gsems = [kw[nm] for nm in sem_names]
        if combine:
            p_h, src_h, dst_h, b_h, out_h, tbl_h = ins
        else:
            tbl_h, src_h, dst_h, out_h = ins
        c = lax.axis_index("c")
        s = lax.axis_index("s")
        wid = c * 16 + s

        di = [pltpu.async_copy(src_h.at[wid], srcv, gsems[0]),
              pltpu.async_copy(dst_h.at[c * NWORK + wid], dstv, gsems[1])]
        for d in di:
            d.wait()

        zeros16 = jnp.zeros((16,), jnp.float32)

        if combine:
            pltpu.sync_copy(b_h, bv)
            nsub = (NP // 16) // CH

            def comb(q, _):
                r0 = s * (NP // 16) + q * CH
                dg = [pltpu.async_copy(p_h.at[0, pl.ds(r0, CH)], rowb.at[0],
                                       gsems[0]),
                      pltpu.async_copy(p_h.at[1, pl.ds(r0, CH)], rowb.at[1],
                                       gsems[1])]
                for d in dg:
                    d.wait()

                def comb_row(r, _):
                    for kk in range(feat // 16):
                        sl = pl.ds(kk * 16, 16)
                        rowb[0, r, sl] = (rowb[0, r, sl] + rowb[1, r, sl]
                                          + bv[pl.ds(kk * 16, 16)])
                    return 0

                lax.fori_loop(0, CH, comb_row, 0)
                pltpu.sync_copy(rowb.at[0], tbl_h.at[pl.ds(c * NP + r0, CH)])
                return 0

            lax.fori_loop(0, nsub, comb, 0)

            def sadd(r, _):
                for kk in range(CH // 16):
                    sl = pl.ds(kk * 16, 16)
                    srcv[r, sl] = srcv[r, sl] + c * NP
                return 0

            lax.fori_loop(0, NCHUNK, sadd, 0)
            table_ref = tbl_h
        else:
            table_ref = tbl_h

        def zrow(r, _):
            for kk in range(feat // 16):
                rowb[0, r, pl.ds(kk * 16, 16)] = zeros16
            return 0

        lax.fori_loop(0, CH, zrow, 0)
        dz = [pltpu.async_copy(rowb.at[0], acc.at[pl.ds(s * rs + j * CH, CH)],
                               ssem)
              for j in range(rs // CH)]
        for d in dz:
            d.wait()
        plsc.subcore_barrier()

        def run_group(base, bufs):
            dgs = [
                pltpu.async_copy(table_ref.at[srcv.at[base + j]],
                                 rowb.at[bufs[j]], gsems[bufs[j]])
                for j in range(len(bufs))
            ]
            return dgs

        def scat_group(base, bufs, dgs):
            dss = []
            for j in range(len(bufs)):
                dgs[j].wait()
                dss.append(
                    pltpu.async_copy(rowb.at[bufs[j]],
                                     acc.at[dstv.at[base + j]],
                                     ssem, add=True))
            return dss

        halves = nbuf // NBUF
        span = NBUF * halves

        def group(g, _):
            base = span * g
            all_dgs = [run_group(base + NBUF * h,
                                 list(range(NBUF * h, NBUF * (h + 1))))
                       for h in range(halves)]
            dss = []
            for h in range(halves):
                dss += scat_group(base + NBUF * h,
                                  list(range(NBUF * h, NBUF * (h + 1))),
                                  all_dgs[h])
            for d in dss:
                d.wait()
            return 0

        nfull = NCHUNK // span
        lax.fori_loop(0, nfull, group, 0)
        tail = NCHUNK - nfull * span
        if tail:
            base = nfull * span
            bufs = list(range(tail))
            dgs = run_group(base, bufs)
            for d in scat_group(base, bufs, dgs):
                d.wait()
        plsc.subcore_barrier()

        do = [pltpu.async_copy(acc.at[pl.ds(s * rs + j * CH, CH)],
                               out_h.at[c, pl.ds(s * rs + j * CH, CH)],
                               ssem)
              for j in range(rs // CH)]
        for d in do:
            d.wait()

    if combine:
        return k(table, src3, dst4, bias)
    return k(table, src3, dst4)


def _tc_call(body, out_shape, *args):
    return pl.pallas_call(
        body, out_shape=jax.ShapeDtypeStruct(out_shape, jnp.float32)
    )(*args)


def _mm_body(x_ref, w_ref, o_ref):
    o_ref[...] = jnp.dot(x_ref[...], w_ref[...], preferred_element_type=jnp.float32)


def _stage2_body(p_ref, wst_ref, bst_ref, wts_ref, bts_ref, w2_ref, b2_ref,
                 w3_ref, o_ref):
    agg_st = p_ref[0, :N, :] + p_ref[1, :N, :]
    agg_ts = p_ref[0, NP:NP + N, :] + p_ref[1, NP:NP + N, :]
    st = jax.nn.relu(
        jnp.dot(agg_st, wst_ref[...], preferred_element_type=jnp.float32)
        + bst_ref[...])
    ts = jax.nn.relu(
        jnp.dot(agg_ts, wts_ref[...], preferred_element_type=jnp.float32)
        + bts_ref[...])
    al = jax.nn.relu(
        jnp.dot(agg_st + agg_ts, w2_ref[...], preferred_element_type=jnp.float32)
        + b2_ref[...])
    w3 = w3_ref[...]
    z = (jnp.dot(st, w3[:32], preferred_element_type=jnp.float32)
         + jnp.dot(ts, w3[32:64], preferred_element_type=jnp.float32)
         + jnp.dot(al, w3[64:], preferred_element_type=jnp.float32))
    o_ref[...] = z


def _final_body(p_ref, b_ref, o_ref):
    h3 = p_ref[0, :N, :] + p_ref[1, :N, :]
    t = h3[:, :40] + b_ref[...]
    m = jnp.max(t, axis=1, keepdims=True)
    e = t - m
    o_ref[...] = e - jnp.log(jnp.sum(jnp.exp(e), axis=1, keepdims=True))


def kernel(x, edge_index, is_reversed, W1, b1, W2, b2, Wst, bst, Wts, bts, W3, b3):
    src = edge_index[0]
    dst = edge_index[1]
    src3 = src.reshape(NWORK, NCHUNK, CH)
    dst4 = jnp.broadcast_to(dst.reshape(NWORK, NCHUNK, CH)[None],
                            (2, NWORK, NCHUNK, CH)).reshape(
                                2 * NWORK, NCHUNK, CH)
    rev_i = is_reversed.astype(jnp.int32)
    dstr4 = jnp.broadcast_to(
        (dst + NP * rev_i).reshape(NWORK, NCHUNK, CH)[None],
        (2, NWORK, NCHUNK, CH)).reshape(2 * NWORK, NCHUNK, CH)

    y1 = _tc_call(_mm_body, (N, 64), x, W1)
    p1 = _seg_sum_sc(y1, src3, dst4, NP, 64, 2 * NBUF)

    p2, _ = _seg_sum_sc(p1, src3, dstr4, 2 * NP, 64, NBUF, bias=b1,
                        combine=True)
    w3p = jnp.pad(W3, ((0, 0), (0, 8)))
    z = _tc_call(_stage2_body, (N, 48), p2,
                 Wst, bst.reshape(1, 32), Wts, bts.reshape(1, 32),
                 W2, b2.reshape(1, 64), w3p)

    p3 = _seg_sum_sc(z, src3, dst4, NP, 48, 2 * NBUF)
    return _tc_call(_final_body, (N, 40), p3, b3.reshape(1, 40))


# speedup vs baseline: 1.0150x; 1.0150x over previous
import functools

import jax
import jax.numpy as jnp
from jax import lax
from jax.experimental import pallas as pl
from jax.experimental.pallas import tpu as pltpu
from jax.experimental.pallas import tpu_sc as plsc

N = 10000
NP = 10240
E = 320000
NWORK = 32
EW = E // NWORK
CH = 80
NCHUNK = EW // CH
NBUF = 5
NGRP = NCHUNK // NBUF


def _seg_sum_sc(table, src3, dst4, rows_out, feat, nbuf, bias=None,
                combine=False):
    mesh = plsc.VectorSubcoreMesh(core_axis_name="c", subcore_axis_name="s",
                                  num_cores=2, num_subcores=16)
    rs = rows_out // 16
    sem_names = ["g%d" % b for b in range(nbuf)]

    out_type = jax.ShapeDtypeStruct((2, rows_out, feat), jnp.float32)
    if combine:
        out_type = (out_type,
                    jax.ShapeDtypeStruct((2 * NP, feat), jnp.float32))

    @functools.partial(
        pl.kernel,
        mesh=mesh,
        out_type=out_type,
        scratch_types=dict(
            srcv=pltpu.VMEM((NCHUNK, CH), jnp.int32),
            dstv=pltpu.VMEM((NCHUNK, CH), jnp.int32),
            rowb=pltpu.VMEM((nbuf, CH, feat), jnp.float32),
            bv=pltpu.VMEM((64,), jnp.float32),
            acc=pltpu.VMEM_SHARED((rows_out, feat), jnp.float32),
            ssem=pltpu.SemaphoreType.DMA,
            **{nm: pltpu.SemaphoreType.DMA for nm in sem_names},
        ),
        compiler_params=pltpu.CompilerParams(use_tc_tiling_on_sc=False),
    )
    def k(*ins, srcv, dstv, rowb, bv, acc, ssem, **kw):
        gsems = [kw[nm] for nm in sem_names]
        if combine:
            p_h, src_h, dst_h, b_h, out_h, tbl_h = ins
        else:
            tbl_h, src_h, dst_h, out_h = ins
        c = lax.axis_index("c")
        s = lax.axis_index("s")
        wid = c * 16 + s

        di = [pltpu.async_copy(src_h.at[wid], srcv, gsems[0]),
              pltpu.async_copy(dst_h.at[c * NWORK + wid], dstv, gsems[1])]
        for d in di:
            d.wait()

        zeros16 = jnp.zeros((16,), jnp.float32)

        if combine:
            pltpu.sync_copy(b_h, bv)
            nsub = (NP // 16) // CH

            def comb(q, _):
                r0 = s * (NP // 16) + q * CH
                dg = [pltpu.async_copy(p_h.at[0, pl.ds(r0, CH)], rowb.at[0],
                                       gsems[0]),
                      pltpu.async_copy(p_h.at[1, pl.ds(r0, CH)], rowb.at[1],
                                       gsems[1])]
                for d in dg:
                    d.wait()

                def comb_row(r, _):
                    for kk in range(feat // 16):
                        sl = pl.ds(kk * 16, 16)
                        rowb[0, r, sl] = (rowb[0, r, sl] + rowb[1, r, sl]
                                          + bv[pl.ds(kk * 16, 16)])
                    return 0

                lax.fori_loop(0, CH, comb_row, 0)
                pltpu.sync_copy(rowb.at[0], tbl_h.at[pl.ds(c * NP + r0, CH)])
                return 0

            lax.fori_loop(0, nsub, comb, 0)

            def sadd(r, _):
                for kk in range(CH // 16):
                    sl = pl.ds(kk * 16, 16)
                    srcv[r, sl] = srcv[r, sl] + c * NP
                return 0

            lax.fori_loop(0, NCHUNK, sadd, 0)
            table_ref = tbl_h
        else:
            table_ref = tbl_h

        def zrow(r, _):
            for kk in range(feat // 16):
                rowb[0, r, pl.ds(kk * 16, 16)] = zeros16
            return 0

        lax.fori_loop(0, CH, zrow, 0)
        dz = [pltpu.async_copy(rowb.at[0], acc.at[pl.ds(s * rs + j * CH, CH)],
                               ssem)
              for j in range(rs // CH)]
        for d in dz:
            d.wait()
        plsc.subcore_barrier()

        def run_group(base, bufs):
            dgs = [
                pltpu.async_copy(table_ref.at[srcv.at[base + j]],
                                 rowb.at[bufs[j]], gsems[bufs[j]])
                for j in range(len(bufs))
            ]
            return dgs

        def scat_group(base, bufs, dgs):
            dss = []
            for j in range(len(bufs)):
                dgs[j].wait()
                dss.append(
                    pltpu.async_copy(rowb.at[bufs[j]],
                                     acc.at[dstv.at[base + j]],
                                     ssem, add=True))
            return dss

        halves = nbuf // NBUF
        span = NBUF * halves

        def group(g, _):
            base = span * g
            all_dgs = [run_group(base + NBUF * h,
                                 list(range(NBUF * h, NBUF * (h + 1))))
                       for h in range(halves)]
            dss = []
            for h in range(halves):
                dss += scat_group(base + NBUF * h,
                                  list(range(NBUF * h, NBUF * (h + 1))),
                                  all_dgs[h])
            for d in dss:
                d.wait()
            return 0

        nfull = NCHUNK // span
        lax.fori_loop(0, nfull, group, 0)
        tail = NCHUNK - nfull * span
        if tail:
            base = nfull * span
            bufs = list(range(tail))
            dgs = run_group(base, bufs)
            for d in scat_group(base, bufs, dgs):
                d.wait()
        plsc.subcore_barrier()

        do = [pltpu.async_copy(acc.at[pl.ds(s * rs + j * CH, CH)],
                               out_h.at[c, pl.ds(s * rs + j * CH, CH)],
                               ssem)
              for j in range(rs // CH)]
        for d in do:
            d.wait()

    if combine:
        return k(table, src3, dst4, bias)
    return k(table, src3, dst4)


def _tc_call(body, out_shape, *args):
    return pl.pallas_call(
        body, out_shape=jax.ShapeDtypeStruct(out_shape, jnp.float32)
    )(*args)


def _mm_body(x_ref, w_ref, o_ref):
    o_ref[...] = jnp.dot(x_ref[...], w_ref[...], preferred_element_type=jnp.float32)


def _comb_body(p_ref, b_ref, o_ref):
    o_ref[...] = p_ref[0, :N, :] + p_ref[1, :N, :] + b_ref[...]


def _stage2_body(p_ref, wst_ref, bst_ref, wts_ref, bts_ref, w2_ref, b2_ref,
                 w3_ref, o_ref):
    agg_st = p_ref[0, :N, :] + p_ref[1, :N, :]
    agg_ts = p_ref[0, NP:NP + N, :] + p_ref[1, NP:NP + N, :]
    st = jax.nn.relu(
        jnp.dot(agg_st, wst_ref[...], preferred_element_type=jnp.float32)
        + bst_ref[...])
    ts = jax.nn.relu(
        jnp.dot(agg_ts, wts_ref[...], preferred_element_type=jnp.float32)
        + bts_ref[...])
    al = jax.nn.relu(
        jnp.dot(agg_st + agg_ts, w2_ref[...], preferred_element_type=jnp.float32)
        + b2_ref[...])
    w3 = w3_ref[...]
    z = (jnp.dot(st, w3[:32], preferred_element_type=jnp.float32)
         + jnp.dot(ts, w3[32:64], preferred_element_type=jnp.float32)
         + jnp.dot(al, w3[64:], preferred_element_type=jnp.float32))
    o_ref[...] = z


def _final_body(p_ref, b_ref, o_ref):
    h3 = p_ref[0, :N, :] + p_ref[1, :N, :]
    t = h3[:, :40] + b_ref[...]
    m = jnp.max(t, axis=1, keepdims=True)
    e = t - m
    o_ref[...] = e - jnp.log(jnp.sum(jnp.exp(e), axis=1, keepdims=True))


def kernel(x, edge_index, is_reversed, W1, b1, W2, b2, Wst, bst, Wts, bts, W3, b3):
    src = edge_index[0]
    dst = edge_index[1]
    src3 = src.reshape(NWORK, NCHUNK, CH)
    dst4 = jnp.broadcast_to(dst.reshape(NWORK, NCHUNK, CH)[None],
                            (2, NWORK, NCHUNK, CH)).reshape(
                                2 * NWORK, NCHUNK, CH)
    rev_i = is_reversed.astype(jnp.int32)
    dstr4 = jnp.broadcast_to(
        (dst + NP * rev_i).reshape(NWORK, NCHUNK, CH)[None],
        (2, NWORK, NCHUNK, CH)).reshape(2 * NWORK, NCHUNK, CH)

    y1 = _tc_call(_mm_body, (N, 64), x, W1)
    p1 = _seg_sum_sc(y1, src3, dst4, NP, 64, 2 * NBUF)

    h1 = _tc_call(_comb_body, (N, 64), p1, b1.reshape(1, 64))
    p2 = _seg_sum_sc(h1, src3, dstr4, 2 * NP, 64, NBUF)
    w3p = jnp.pad(W3, ((0, 0), (0, 8)))
    z = _tc_call(_stage2_body, (N, 48), p2,
                 Wst, bst.reshape(1, 32), Wts, bts.reshape(1, 32),
                 W2, b2.reshape(1, 64), w3p)

    p3 = _seg_sum_sc(z, src3, dst4, NP, 48, 2 * NBUF)
    return _tc_call(_final_body, (N, 40), p3, b3.reshape(1, 40))


# speedup vs baseline: 1.0449x; 1.0295x over previous
import functools

import jax
import jax.numpy as jnp
from jax import lax
from jax.experimental import pallas as pl
from jax.experimental.pallas import tpu as pltpu
from jax.experimental.pallas import tpu_sc as plsc

N = 10000
NP = 10240
E = 320000
NWORK = 32
EW = E // NWORK
CH = 80
NCHUNK = EW // CH
NBUF = 5
NGRP = NCHUNK // NBUF


def _seg_sum_sc(table, src3, dst4, rows_out, feat, nbuf, bias=None,
                combine=False):
    mesh = plsc.VectorSubcoreMesh(core_axis_name="c", subcore_axis_name="s",
                                  num_cores=2, num_subcores=16)
    rs = rows_out // 16
    sem_names = ["g%d" % b for b in range(nbuf)]

    out_type = jax.ShapeDtypeStruct((2, rows_out, feat), jnp.float32)
    if combine:
        out_type = (out_type,
                    jax.ShapeDtypeStruct((2 * NP, feat), jnp.float32))

    @functools.partial(
        pl.kernel,
        mesh=mesh,
        out_type=out_type,
        scratch_types=dict(
            srcv=pltpu.VMEM((NCHUNK, CH), jnp.int32),
            dstv=pltpu.VMEM((NCHUNK, CH), jnp.int32),
            rowb=pltpu.VMEM((nbuf, CH, feat), jnp.float32),
            bv=pltpu.VMEM((64,), jnp.float32),
            acc=pltpu.VMEM_SHARED((rows_out, feat), jnp.float32),
            ssem=pltpu.SemaphoreType.DMA,
            **{nm: pltpu.SemaphoreType.DMA for nm in sem_names},
        ),
        compiler_params=pltpu.CompilerParams(use_tc_tiling_on_sc=False),
    )
    def k(*ins, srcv, dstv, rowb, bv, acc, ssem, **kw):
        gsems = [kw[nm] for nm in sem_names]
        if combine:
            p_h, src_h, dst_h, b_h, out_h, tbl_h = ins
        else:
            tbl_h, src_h, dst_h, out_h = ins
        c = lax.axis_index("c")
        s = lax.axis_index("s")
        wid = c * 16 + s

        di = [pltpu.async_copy(src_h.at[wid], srcv, gsems[0]),
              pltpu.async_copy(dst_h.at[wid], dstv, gsems[1])]
        for d in di:
            d.wait()

        zeros16 = jnp.zeros((16,), jnp.float32)

        if combine:
            pltpu.sync_copy(b_h, bv)
            nsub = (NP // 16) // CH

            def comb(q, _):
                r0 = s * (NP // 16) + q * CH
                dg = [pltpu.async_copy(p_h.at[0, pl.ds(r0, CH)], rowb.at[0],
                                       gsems[0]),
                      pltpu.async_copy(p_h.at[1, pl.ds(r0, CH)], rowb.at[1],
                                       gsems[1])]
                for d in dg:
                    d.wait()

                def comb_row(r, _):
                    for kk in range(feat // 16):
                        sl = pl.ds(kk * 16, 16)
                        rowb[0, r, sl] = (rowb[0, r, sl] + rowb[1, r, sl]
                                          + bv[pl.ds(kk * 16, 16)])
                    return 0

                lax.fori_loop(0, CH, comb_row, 0)
                pltpu.sync_copy(rowb.at[0], tbl_h.at[pl.ds(c * NP + r0, CH)])
                return 0

            lax.fori_loop(0, nsub, comb, 0)

            def sadd(r, _):
                for kk in range(CH // 16):
                    sl = pl.ds(kk * 16, 16)
                    srcv[r, sl] = srcv[r, sl] + c * NP
                return 0

            lax.fori_loop(0, NCHUNK, sadd, 0)
            table_ref = tbl_h
        else:
            table_ref = tbl_h

        def zrow(r, _):
            for kk in range(feat // 16):
                rowb[0, r, pl.ds(kk * 16, 16)] = zeros16
            return 0

        lax.fori_loop(0, CH, zrow, 0)
        dz = [pltpu.async_copy(rowb.at[0], acc.at[pl.ds(s * rs + j * CH, CH)],
                               ssem)
              for j in range(rs // CH)]
        for d in dz:
            d.wait()
        plsc.subcore_barrier()

        def run_group(base, bufs):
            dgs = [
                pltpu.async_copy(table_ref.at[srcv.at[base + j]],
                                 rowb.at[bufs[j]], gsems[bufs[j]])
                for j in range(len(bufs))
            ]
            return dgs

        def scat_group(base, bufs, dgs):
            dss = []
            for j in range(len(bufs)):
                dgs[j].wait()
                dss.append(
                    pltpu.async_copy(rowb.at[bufs[j]],
                                     acc.at[dstv.at[base + j]],
                                     ssem, add=True))
            return dss

        halves = nbuf // NBUF
        span = NBUF * halves

        def group(g, _):
            base = span * g
            all_dgs = [run_group(base + NBUF * h,
                                 list(range(NBUF * h, NBUF * (h + 1))))
                       for h in range(halves)]
            dss = []
            for h in range(halves):
                dss += scat_group(base + NBUF * h,
                                  list(range(NBUF * h, NBUF * (h + 1))),
                                  all_dgs[h])
            for d in dss:
                d.wait()
            return 0

        nfull = NCHUNK // span
        lax.fori_loop(0, nfull, group, 0)
        tail = NCHUNK - nfull * span
        if tail:
            base = nfull * span
            bufs = list(range(tail))
            dgs = run_group(base, bufs)
            for d in scat_group(base, bufs, dgs):
                d.wait()
        plsc.subcore_barrier()

        do = [pltpu.async_copy(acc.at[pl.ds(s * rs + j * CH, CH)],
                               out_h.at[c, pl.ds(s * rs + j * CH, CH)],
                               ssem)
              for j in range(rs // CH)]
        for d in do:
            d.wait()

    if combine:
        return k(table, src3, dst4, bias)
    return k(table, src3, dst4)


def _tc_call(body, out_shape, *args):
    return pl.pallas_call(
        body, out_shape=jax.ShapeDtypeStruct(out_shape, jnp.float32)
    )(*args)


def _mm_body(x_ref, w_ref, o_ref):
    o_ref[...] = jnp.dot(x_ref[...], w_ref[...], preferred_element_type=jnp.float32)


def _comb_body(p_ref, b_ref, o_ref):
    o_ref[...] = p_ref[0, :N, :] + p_ref[1, :N, :] + b_ref[...]


def _stage2_body(p_ref, wst_ref, bst_ref, wts_ref, bts_ref, w2_ref, b2_ref,
                 w3_ref, o_ref):
    agg_st = p_ref[0, :N, :] + p_ref[1, :N, :]
    agg_ts = p_ref[0, NP:NP + N, :] + p_ref[1, NP:NP + N, :]
    st = jax.nn.relu(
        jnp.dot(agg_st, wst_ref[...], preferred_element_type=jnp.float32)
        + bst_ref[...])
    ts = jax.nn.relu(
        jnp.dot(agg_ts, wts_ref[...], preferred_element_type=jnp.float32)
        + bts_ref[...])
    al = jax.nn.relu(
        jnp.dot(agg_st + agg_ts, w2_ref[...], preferred_element_type=jnp.float32)
        + b2_ref[...])
    w3 = w3_ref[...]
    z = (jnp.dot(st, w3[:32], preferred_element_type=jnp.float32)
         + jnp.dot(ts, w3[32:64], preferred_element_type=jnp.float32)
         + jnp.dot(al, w3[64:], preferred_element_type=jnp.float32))
    o_ref[...] = z


def _final_body(p_ref, b_ref, o_ref):
    h3 = p_ref[0, :N, :] + p_ref[1, :N, :]
    t = h3[:, :40] + b_ref[...]
    m = jnp.max(t, axis=1, keepdims=True)
    e = t - m
    o_ref[...] = e - jnp.log(jnp.sum(jnp.exp(e), axis=1, keepdims=True))


def kernel(x, edge_index, is_reversed, W1, b1, W2, b2, Wst, bst, Wts, bts, W3, b3):
    src = edge_index[0]
    dst = edge_index[1]
    src3 = src.reshape(NWORK, NCHUNK, CH)
    dst4 = dst.reshape(NWORK, NCHUNK, CH)
    rev_i = is_reversed.astype(jnp.int32)
    dstr4 = (dst + NP * rev_i).reshape(NWORK, NCHUNK, CH)

    y1 = _tc_call(_mm_body, (N, 64), x, W1)
    p1 = _seg_sum_sc(y1, src3, dst4, NP, 64, 2 * NBUF)

    h1 = _tc_call(_comb_body, (N, 64), p1, b1.reshape(1, 64))
    p2 = _seg_sum_sc(h1, src3, dstr4, 2 * NP, 64, NBUF)
    w3p = jnp.pad(W3, ((0, 0), (0, 8)))
    z = _tc_call(_stage2_body, (N, 48), p2,
                 Wst, bst.reshape(1, 32), Wts, bts.reshape(1, 32),
                 W2, b2.reshape(1, 64), w3p)

    p3 = _seg_sum_sc(z, src3, dst4, NP, 48, 2 * NBUF)
    return _tc_call(_final_body, (N, 40), p3, b3.reshape(1, 40))
